# baseline (speedup 1.0000x reference)
import jax
import jax.numpy as jnp
from jax import lax
from jax.experimental import pallas as pl
from jax.experimental.pallas import tpu as pltpu
from jax.experimental.pallas import tpu_sc as plsc

BATCH = 4096
HIST = 200
FEATURE_DIM = 100000
EMBED = 128
NUM_WORKERS = 32
BPW = BATCH // NUM_WORKERS
ZROW = FEATURE_DIM + 1
IPW = BPW * HIST
CH = IPW // BPW
NRING = 4
NIT = CH // NRING
RSTEP = -(-(IPW // 16) // (NIT - 1))


def _sc_body(xb_hbm, xq_hbm, neg_hbm, tab_hbm, q_out, ub_out, nub_out,
             xbuf, nbuf, ebufs, qidx, sem, gsems, wsems):
    wid = lax.axis_index("s") * 2 + lax.axis_index("c")
    base = wid * BPW

    pltpu.sync_copy(xb_hbm.at[pl.ds(base * HIST, IPW)], xbuf)
    pltpu.sync_copy(neg_hbm.at[pl.ds(base * HIST, IPW)], nbuf)
    pltpu.sync_copy(xq_hbm.at[pl.ds(base, BPW)], qidx)

    zeros16 = jnp.zeros((16,), jnp.int32)
    zrow16 = jnp.full((16,), ZROW, jnp.int32)

    def remap16(ref, o):
        v = ref[pl.ds(o, 16)]
        ref[pl.ds(o, 16)] = jnp.where(v > zeros16, v, zrow16)

    def remap_x(s, carry):
        remap16(xbuf, s * 16)
        return carry

    lax.fori_loop(0, IPW // 16, remap_x, 0)

    pltpu.async_copy(tab_hbm.at[qidx], ebufs[0], sem).wait()
    pltpu.sync_copy(ebufs[0], q_out.at[pl.ds(base, BPW)])

    def start_g(idx_ref, t, k):
        pltpu.async_copy(tab_hbm.at[idx_ref.at[pl.ds(t * BPW, BPW)]],
                         ebufs[k], gsems[k])

    def drain_g(k):
        pltpu.make_async_copy(tab_hbm.at[pl.ds(0, BPW), :], ebufs[k],
                              gsems[k]).wait()

    wbase = base * HIST

    def step(i, carry):
        for k in range(NRING):
            t = NRING * i + k
            drain_g(k)

            @pl.when(t < CH)
            def _(t=t, k=k):
                pltpu.async_copy(
                    ebufs[k], ub_out.at[pl.ds(wbase + t * BPW, BPW), :],
                    wsems[k])

            @pl.when(t >= CH)
            def _(t=t, k=k):
                pltpu.async_copy(
                    ebufs[k],
                    nub_out.at[pl.ds(wbase + (t - CH) * BPW, BPW), :],
                    wsems[k])

        for s in range(RSTEP):
            o = (RSTEP * i + s) * 16

            @pl.when(o < IPW)
            def _(o=o):
                remap16(nbuf, o)

        for k in range(NRING):
            t = NRING * i + k
            pltpu.make_async_copy(ebufs[k], ub_out.at[pl.ds(wbase, BPW), :],
                                  wsems[k]).wait()
            tn = t + NRING

            @pl.when(tn < CH)
            def _(tn=tn, k=k):
                start_g(xbuf, tn, k)

            @pl.when((tn >= CH) & (tn < 2 * CH))
            def _(tn=tn, k=k):
                start_g(nbuf, tn - CH, k)

        return carry

    for k in range(NRING):
        start_g(xbuf, k, k)
    lax.fori_loop(0, 2 * NIT, step, 0)


def _tc_len_body(x_ref, o_ref):
    m = jnp.where(x_ref[...] > 0, jnp.int32(1), jnp.int32(0))
    o_ref[...] = jnp.sum(m, axis=1)


@jax.jit
def _impl(x, neg_x, table):
    tab2 = jnp.concatenate(
        [table, jnp.zeros((1, EMBED), jnp.float32)], axis=0)
    xb2d = x[:, :HIST]
    xb = xb2d.reshape(-1)
    xq = x[:, HIST]
    negf = neg_x.reshape(-1)

    fn = pl.kernel(
        _sc_body,
        out_type=(
            jax.ShapeDtypeStruct((BATCH, EMBED), jnp.float32),
            jax.ShapeDtypeStruct((BATCH * HIST, EMBED), jnp.float32),
            jax.ShapeDtypeStruct((BATCH * HIST, EMBED), jnp.float32),
        ),
        mesh=plsc.VectorSubcoreMesh(core_axis_name="c", subcore_axis_name="s"),
        scratch_types=[
            pltpu.VMEM((IPW,), jnp.int32),
            pltpu.VMEM((IPW,), jnp.int32),
            [pltpu.VMEM((BPW, EMBED), jnp.float32) for _ in range(NRING)],
            pltpu.VMEM((BPW,), jnp.int32),
            pltpu.SemaphoreType.DMA,
            [pltpu.SemaphoreType.DMA for _ in range(NRING)],
            [pltpu.SemaphoreType.DMA for _ in range(NRING)],
        ],
    )
    q, ub, nub = fn(xb, xq, negf, tab2)

    bl = pl.pallas_call(
        _tc_len_body,
        grid=(8,),
        in_specs=[pl.BlockSpec((BATCH // 8, HIST), lambda i: (i, 0))],
        out_specs=pl.BlockSpec((BATCH // 8,), lambda i: (i,)),
        out_shape=jax.ShapeDtypeStruct((BATCH,), jnp.int32),
    )(xb2d)

    return (q.reshape(BATCH, 1, EMBED), ub.reshape(BATCH, HIST, EMBED), bl,
            nub.reshape(BATCH, HIST, EMBED))


def kernel(x, neg_x, table):
    return _impl(x, neg_x, table)


# speedup vs baseline: 1.0018x; 1.0018x over previous
import jax
import jax.numpy as jnp
from jax import lax
from jax.experimental import pallas as pl
from jax.experimental.pallas import tpu as pltpu
from jax.experimental.pallas import tpu_sc as plsc

BATCH = 4096
HIST = 200
FEATURE_DIM = 100000
EMBED = 128
NUM_WORKERS = 32
BPW = BATCH // NUM_WORKERS
ZROW = FEATURE_DIM + 1
IPW = BPW * HIST
CH = IPW // BPW
NRING = 4
NIT = CH // NRING
RSTEP = (IPW // 16) // NIT


def _sc_body(xb_hbm, xq_hbm, neg_hbm, tab_hbm, q_out, ub_out, nub_out,
             xbuf, nbuf, ebufs, qidx, sem, gsems, wsems):
    wid = lax.axis_index("s") * 2 + lax.axis_index("c")
    base = wid * BPW

    pltpu.sync_copy(xb_hbm.at[pl.ds(base * HIST, IPW)], xbuf)
    pltpu.sync_copy(neg_hbm.at[pl.ds(base * HIST, IPW)], nbuf)
    pltpu.sync_copy(xq_hbm.at[pl.ds(base, BPW)], qidx)

    zeros16 = jnp.zeros((16,), jnp.int32)
    zrow16 = jnp.full((16,), ZROW, jnp.int32)

    def remap16(ref, o):
        v = ref[pl.ds(o, 16)]
        ref[pl.ds(o, 16)] = jnp.where(v > zeros16, v, zrow16)

    def remap_x(s, carry):
        remap16(xbuf, s * 16)
        return carry

    lax.fori_loop(0, IPW // 16, remap_x, 0)

    pltpu.async_copy(tab_hbm.at[qidx], ebufs[0], sem).wait()
    pltpu.sync_copy(ebufs[0], q_out.at[pl.ds(base, BPW)])

    def start_g(idx_ref, t, k):
        pltpu.async_copy(tab_hbm.at[idx_ref.at[pl.ds(t * BPW, BPW)]],
                         ebufs[k], gsems[k])

    def drain_g(k):
        pltpu.make_async_copy(tab_hbm.at[pl.ds(0, BPW), :], ebufs[k],
                              gsems[k]).wait()

    def emit(idx_ref, out_ref, wbase, hidden):
        for k in range(NRING):
            start_g(idx_ref, k, k)

        def step(i, carry):
            for k in range(NRING):
                t = NRING * i + k
                drain_g(k)
                pltpu.async_copy(
                    ebufs[k], out_ref.at[pl.ds(wbase + t * BPW, BPW), :],
                    wsems[k])

            hidden(i)

            for k in range(NRING):
                t = NRING * i + k
                pltpu.make_async_copy(
                    ebufs[k], out_ref.at[pl.ds(wbase + t * BPW, BPW), :],
                    wsems[k]).wait()

                @pl.when(t + NRING < CH)
                def _(t=t, k=k):
                    start_g(idx_ref, t + NRING, k)

            return carry

        lax.fori_loop(0, NIT, step, 0)

    def user_hidden(i):
        for s in range(RSTEP):
            remap16(nbuf, (RSTEP * i + s) * 16)

    emit(xbuf, ub_out, base * HIST, user_hidden)
    emit(nbuf, nub_out, base * HIST, lambda i: None)


def _tc_len_body(x_ref, o_ref):
    m = jnp.where(x_ref[...] > 0, jnp.int32(1), jnp.int32(0))
    o_ref[...] = jnp.sum(m, axis=1)


@jax.jit
def _impl(x, neg_x, table):
    tab2 = jnp.concatenate(
        [table, jnp.zeros((1, EMBED), jnp.float32)], axis=0)
    xb2d = x[:, :HIST]
    xb = xb2d.reshape(-1)
    xq = x[:, HIST]
    negf = neg_x.reshape(-1)

    fn = pl.kernel(
        _sc_body,
        out_type=(
            jax.ShapeDtypeStruct((BATCH, EMBED), jnp.float32),
            jax.ShapeDtypeStruct((BATCH * HIST, EMBED), jnp.float32),
            jax.ShapeDtypeStruct((BATCH * HIST, EMBED), jnp.float32),
        ),
        mesh=plsc.VectorSubcoreMesh(core_axis_name="c", subcore_axis_name="s"),
        scratch_types=[
            pltpu.VMEM((IPW,), jnp.int32),
            pltpu.VMEM((IPW,), jnp.int32),
            [pltpu.VMEM((BPW, EMBED), jnp.float32) for _ in range(NRING)],
            pltpu.VMEM((BPW,), jnp.int32),
            pltpu.SemaphoreType.DMA,
            [pltpu.SemaphoreType.DMA for _ in range(NRING)],
            [pltpu.SemaphoreType.DMA for _ in range(NRING)],
        ],
    )
    q, ub, nub = fn(xb, xq, negf, tab2)

    bl = pl.pallas_call(
        _tc_len_body,
        grid=(8,),
        in_specs=[pl.BlockSpec((BATCH // 8, HIST), lambda i: (i, 0))],
        out_specs=pl.BlockSpec((BATCH // 8,), lambda i: (i,)),
        out_shape=jax.ShapeDtypeStruct((BATCH,), jnp.int32),
    )(xb2d)

    return (q.reshape(BATCH, 1, EMBED), ub.reshape(BATCH, HIST, EMBED), bl,
            nub.reshape(BATCH, HIST, EMBED))


def kernel(x, neg_x, table):
    return _impl(x, neg_x, table)
